# baseline (speedup 1.0000x reference)
import functools

import jax
import jax.numpy as jnp
from jax import lax
from jax.experimental import pallas as pl
from jax.experimental.pallas import tpu as pltpu
from jax.experimental.pallas import tpu_sc as plsc

_B = 128
_N = 32768
_L = 16
_NSL = _N // _L
_NC = 2
_NS = 16
_NW = _NC * _NS
_RPW = _B // _NW
_BISECT_ITERS = 24

_NEG = -3.0e38


def _sc_body(x_hbm, out_hbm, row_v, cand_v):
    cid = lax.axis_index("c")
    sid = lax.axis_index("s")
    wid = sid * _NC + cid

    for r in range(_RPW):
        row = wid * _RPW + r
        pltpu.sync_copy(x_hbm.at[row], row_v)

        def max_body(i, m):
            return jnp.maximum(m, row_v[pl.ds(i * _L, _L)])

        m16 = lax.fori_loop(
            0, _NSL, max_body, jnp.full((_L,), _NEG, jnp.float32)
        )
        mx = jnp.max(m16)
        thr16 = jnp.full((_L,), mx - 1.0, jnp.float32)

        def cmp_body(i, off):
            v = row_v[pl.ds(i * _L, _L)]
            msk = v > thr16
            plsc.store_compressed(cand_v.at[pl.ds(off, _L)], v, mask=msk)
            cnt16 = plsc.all_reduce_population_count(msk)
            return off + cnt16[0]

        n_cand = lax.fori_loop(0, _NSL, cmp_body, 0)
        cand_v[pl.ds(n_cand, _L)] = jnp.full((_L,), _NEG, jnp.float32)
        n_csl = (n_cand + _L - 1) // _L

        def bis_body(_, lohi):
            lo, hi = lohi
            mid = 0.5 * (lo + hi)

            def f_body(i, acc):
                v = cand_v[pl.ds(i * _L, _L)]
                return acc + jnp.maximum(v - mid, 0.0)

            s16 = lax.fori_loop(0, n_csl, f_body, jnp.zeros((_L,), jnp.float32))
            fs16 = jnp.full((_L,), jnp.sum(s16) - 1.0, jnp.float32)
            pred = fs16 >= 0.0
            return jnp.where(pred, mid, lo), jnp.where(pred, hi, mid)

        hi0 = jnp.full((_L,), mx, jnp.float32)
        lo16, hi16 = lax.fori_loop(0, _BISECT_ITERS, bis_body, (thr16, hi0))

        def ref_body(i, carry):
            sa, ca = carry
            v = cand_v[pl.ds(i * _L, _L)]
            msk = v > lo16
            return (
                sa + jnp.where(msk, v, 0.0),
                ca + jnp.where(msk, 1.0, 0.0),
            )

        z16 = jnp.zeros((_L,), jnp.float32)
        s16, c16 = lax.fori_loop(0, n_csl, ref_body, (z16, z16))
        num16 = jnp.full((_L,), jnp.sum(s16) - 1.0, jnp.float32)
        den16 = jnp.full((_L,), jnp.sum(c16), jnp.float32)
        theta16 = num16 / den16

        def out_body(i, carry):
            v = row_v[pl.ds(i * _L, _L)]
            row_v[pl.ds(i * _L, _L)] = jnp.maximum(v - theta16, 0.0)
            return carry

        lax.fori_loop(0, _NSL, out_body, 0)
        pltpu.sync_copy(row_v, out_hbm.at[row])


def kernel(inputs):
    mesh = plsc.VectorSubcoreMesh(core_axis_name="c", subcore_axis_name="s")
    f = functools.partial(
        pl.kernel,
        out_type=jax.ShapeDtypeStruct((_B, _N), jnp.float32),
        mesh=mesh,
        compiler_params=pltpu.CompilerParams(needs_layout_passes=False),
        scratch_types=[
            pltpu.VMEM((_N,), jnp.float32),
            pltpu.VMEM((_N + _L,), jnp.float32),
        ],
    )(_sc_body)
    return f(inputs)


# speedup vs baseline: 2.1327x; 2.1327x over previous
import functools

import jax
import jax.numpy as jnp
from jax import lax
from jax.experimental import pallas as pl
from jax.experimental.pallas import tpu as pltpu
from jax.experimental.pallas import tpu_sc as plsc

_B = 128
_N = 32768
_L = 16
_NSL = _N // _L
_NC = 2
_NS = 16
_NW = _NC * _NS
_RPW = _B // _NW
_BISECT_ITERS = 24
_UNROLL = 8

_NEG = -3.0e38


def _row_sparsemax(row_v, cand_v):
    iota16 = lax.iota(jnp.int32, _L)

    def max_body(i, m):
        vs = [row_v[pl.ds((i + k) * _L, _L)] for k in range(_UNROLL)]
        t01 = jnp.maximum(vs[0], vs[1])
        t23 = jnp.maximum(vs[2], vs[3])
        t45 = jnp.maximum(vs[4], vs[5])
        t67 = jnp.maximum(vs[6], vs[7])
        t = jnp.maximum(jnp.maximum(t01, t23), jnp.maximum(t45, t67))
        return jnp.maximum(m, t)

    m16 = plsc.parallel_loop(
        0, _NSL, _UNROLL, unroll=2,
        carry=jnp.full((_L,), _NEG, jnp.float32),
    )(max_body)
    mx = jnp.max(m16)
    thr16 = jnp.full((_L,), mx - 1.0, jnp.float32)

    def cmp_body(i, cnt16):
        for k in range(_UNROLL):
            v = row_v[pl.ds((i + k) * _L, _L)]
            msk = v > thr16
            idx16 = cnt16 * _L + iota16
            plsc.store_scatter(cand_v, [idx16], v, mask=msk)
            cnt16 = cnt16 + jnp.where(msk, 1, 0).astype(jnp.int32)
        return cnt16

    cnt16 = plsc.parallel_loop(
        0, _NSL, _UNROLL, unroll=1,
        carry=jnp.zeros((_L,), jnp.int32),
    )(cmp_body)
    maxn = jnp.max(cnt16)

    def clr_body(j, carry):
        j16 = jnp.full((_L,), j, jnp.int32)
        cur = cand_v[pl.ds(j * _L, _L)]
        cand_v[pl.ds(j * _L, _L)] = jnp.where(
            j16 < cnt16, cur, jnp.full((_L,), _NEG, jnp.float32)
        )
        return carry

    lax.fori_loop(0, maxn, clr_body, 0)

    def bis_body(_, lohi):
        lo, hi = lohi
        mid = 0.5 * (lo + hi)

        def f_body(j, acc):
            v = cand_v[pl.ds(j * _L, _L)]
            return acc + jnp.maximum(v - mid, 0.0)

        s16 = lax.fori_loop(0, maxn, f_body, jnp.zeros((_L,), jnp.float32))
        fs16 = jnp.full((_L,), jnp.sum(s16) - 1.0, jnp.float32)
        pred = fs16 >= 0.0
        return jnp.where(pred, mid, lo), jnp.where(pred, hi, mid)

    hi0 = jnp.full((_L,), mx, jnp.float32)
    lo16, _ = lax.fori_loop(0, _BISECT_ITERS, bis_body, (thr16, hi0))

    def ref_body(j, carry):
        sa, ca = carry
        v = cand_v[pl.ds(j * _L, _L)]
        msk = v > lo16
        return (
            sa + jnp.where(msk, v, 0.0),
            ca + jnp.where(msk, 1.0, 0.0),
        )

    z16 = jnp.zeros((_L,), jnp.float32)
    s16, c16 = lax.fori_loop(0, maxn, ref_body, (z16, z16))
    num16 = jnp.full((_L,), jnp.sum(s16) - 1.0, jnp.float32)
    den16 = jnp.full((_L,), jnp.sum(c16), jnp.float32)
    theta16 = num16 / den16

    def out_body(i):
        for k in range(_UNROLL):
            v = row_v[pl.ds((i + k) * _L, _L)]
            row_v[pl.ds((i + k) * _L, _L)] = jnp.maximum(v - theta16, 0.0)

    plsc.parallel_loop(0, _NSL, _UNROLL, unroll=2)(out_body)


def _sc_body(x_hbm, out_hbm, buf0, buf1, cand_v, si0, si1, so0, so1):
    cid = lax.axis_index("c")
    sid = lax.axis_index("s")
    wid = sid * _NC + cid
    base = wid * _RPW

    bufs = (buf0, buf1)
    isems = (si0, si1)
    osems = (so0, so1)
    cp_in = [None] * _RPW
    cp_out = [None] * _RPW
    cp_in[0] = pltpu.async_copy(x_hbm.at[base], bufs[0], isems[0])

    for r in range(_RPW):
        b = r & 1
        cp_in[r].wait()
        if r + 1 < _RPW:
            if r >= 1:
                cp_out[r - 1].wait()
            cp_in[r + 1] = pltpu.async_copy(
                x_hbm.at[base + r + 1], bufs[1 - b], isems[1 - b]
            )
        _row_sparsemax(bufs[b], cand_v)
        cp_out[r] = pltpu.async_copy(bufs[b], out_hbm.at[base + r], osems[b])

    cp_out[_RPW - 2].wait()
    cp_out[_RPW - 1].wait()


def kernel(inputs):
    mesh = plsc.VectorSubcoreMesh(core_axis_name="c", subcore_axis_name="s")
    f = functools.partial(
        pl.kernel,
        out_type=jax.ShapeDtypeStruct((_B, _N), jnp.float32),
        mesh=mesh,
        compiler_params=pltpu.CompilerParams(needs_layout_passes=False),
        scratch_types=[
            pltpu.VMEM((_N,), jnp.float32),
            pltpu.VMEM((_N,), jnp.float32),
            pltpu.VMEM((_N,), jnp.float32),
            pltpu.SemaphoreType.DMA,
            pltpu.SemaphoreType.DMA,
            pltpu.SemaphoreType.DMA,
            pltpu.SemaphoreType.DMA,
        ],
    )(_sc_body)
    return f(inputs)


# speedup vs baseline: 3.7797x; 1.7723x over previous
import functools

import jax
import jax.numpy as jnp
from jax import lax
from jax.experimental import pallas as pl
from jax.experimental.pallas import tpu as pltpu
from jax.experimental.pallas import tpu_sc as plsc

_B = 128
_N = 32768
_L = 16
_NSL = _N // _L
_NC = 2
_NS = 16
_NW = _NC * _NS
_RPW = _B // _NW
_BISECT_ITERS = 24
_UNROLL = 8

_NEG = -3.0e38


def _row_sparsemax(row_v, cand_v):
    iota16 = lax.iota(jnp.int32, _L)

    def max_body(i, m):
        vs = [row_v[pl.ds((i + k) * _L, _L)] for k in range(_UNROLL)]
        t01 = jnp.maximum(vs[0], vs[1])
        t23 = jnp.maximum(vs[2], vs[3])
        t45 = jnp.maximum(vs[4], vs[5])
        t67 = jnp.maximum(vs[6], vs[7])
        t = jnp.maximum(jnp.maximum(t01, t23), jnp.maximum(t45, t67))
        return jnp.maximum(m, t)

    m16 = plsc.parallel_loop(
        0, _NSL, _UNROLL, unroll=2,
        carry=jnp.full((_L,), _NEG, jnp.float32),
    )(max_body)
    mx = jnp.max(m16)
    thr16 = jnp.full((_L,), mx - 1.0, jnp.float32)

    def cmp_body(i, cnt16):
        vs, ms, cs = [], [], []
        c = cnt16
        for k in range(_UNROLL):
            v = row_v[pl.ds((i + k) * _L, _L)]
            msk = v > thr16
            vs.append(v)
            ms.append(msk)
            cs.append(c)
            c = c + msk.astype(jnp.int32)
        anym = ms[0]
        for k in range(1, _UNROLL):
            anym = jnp.logical_or(anym, ms[k])

        @pl.when(jnp.any(anym))
        def _():
            for k in range(_UNROLL):
                idx16 = cs[k] * _L + iota16
                plsc.store_scatter(cand_v, [idx16], vs[k], mask=ms[k])

        return c

    cnt16 = plsc.parallel_loop(
        0, _NSL, _UNROLL, unroll=1,
        carry=jnp.zeros((_L,), jnp.int32),
    )(cmp_body)
    maxn = jnp.max(cnt16)

    def clr_body(j, carry):
        j16 = jnp.full((_L,), j, jnp.int32)
        cur = cand_v[pl.ds(j * _L, _L)]
        cand_v[pl.ds(j * _L, _L)] = jnp.where(
            j16 < cnt16, cur, jnp.full((_L,), _NEG, jnp.float32)
        )
        return carry

    lax.fori_loop(0, maxn, clr_body, 0)

    def bis_body(_, lohi):
        lo, hi = lohi
        mid = 0.5 * (lo + hi)

        def f_body(j, acc):
            v = cand_v[pl.ds(j * _L, _L)]
            return acc + jnp.maximum(v - mid, 0.0)

        s16 = lax.fori_loop(0, maxn, f_body, jnp.zeros((_L,), jnp.float32))
        fs16 = jnp.full((_L,), jnp.sum(s16) - 1.0, jnp.float32)
        pred = fs16 >= 0.0
        return jnp.where(pred, mid, lo), jnp.where(pred, hi, mid)

    hi0 = jnp.full((_L,), mx, jnp.float32)
    lo16, _ = lax.fori_loop(0, _BISECT_ITERS, bis_body, (thr16, hi0))

    def ref_body(j, carry):
        sa, ca = carry
        v = cand_v[pl.ds(j * _L, _L)]
        msk = v > lo16
        return (
            sa + jnp.where(msk, v, 0.0),
            ca + jnp.where(msk, 1.0, 0.0),
        )

    z16 = jnp.zeros((_L,), jnp.float32)
    s16, c16 = lax.fori_loop(0, maxn, ref_body, (z16, z16))
    num16 = jnp.full((_L,), jnp.sum(s16) - 1.0, jnp.float32)
    den16 = jnp.full((_L,), jnp.sum(c16), jnp.float32)
    theta16 = num16 / den16

    def out_body(i):
        for k in range(_UNROLL):
            v = row_v[pl.ds((i + k) * _L, _L)]
            row_v[pl.ds((i + k) * _L, _L)] = jnp.maximum(v - theta16, 0.0)

    plsc.parallel_loop(0, _NSL, _UNROLL, unroll=2)(out_body)


def _sc_body(x_hbm, out_hbm, buf0, buf1, cand_v, si0, si1, so0, so1):
    cid = lax.axis_index("c")
    sid = lax.axis_index("s")
    wid = sid * _NC + cid
    base = wid * _RPW

    bufs = (buf0, buf1)
    isems = (si0, si1)
    osems = (so0, so1)
    cp_in = [None] * _RPW
    cp_out = [None] * _RPW
    cp_in[0] = pltpu.async_copy(x_hbm.at[base], bufs[0], isems[0])

    for r in range(_RPW):
        b = r & 1
        cp_in[r].wait()
        if r + 1 < _RPW:
            if r >= 1:
                cp_out[r - 1].wait()
            cp_in[r + 1] = pltpu.async_copy(
                x_hbm.at[base + r + 1], bufs[1 - b], isems[1 - b]
            )
        _row_sparsemax(bufs[b], cand_v)
        cp_out[r] = pltpu.async_copy(bufs[b], out_hbm.at[base + r], osems[b])

    cp_out[_RPW - 2].wait()
    cp_out[_RPW - 1].wait()


def kernel(inputs):
    mesh = plsc.VectorSubcoreMesh(core_axis_name="c", subcore_axis_name="s")
    f = functools.partial(
        pl.kernel,
        out_type=jax.ShapeDtypeStruct((_B, _N), jnp.float32),
        mesh=mesh,
        compiler_params=pltpu.CompilerParams(needs_layout_passes=False),
        scratch_types=[
            pltpu.VMEM((_N,), jnp.float32),
            pltpu.VMEM((_N,), jnp.float32),
            pltpu.VMEM((_N,), jnp.float32),
            pltpu.SemaphoreType.DMA,
            pltpu.SemaphoreType.DMA,
            pltpu.SemaphoreType.DMA,
            pltpu.SemaphoreType.DMA,
        ],
    )(_sc_body)
    return f(inputs)
